# Initial kernel scaffold; baseline (speedup 1.0000x reference)
import jax
import jax.numpy as jnp
from jax.experimental import pallas as pl


def kernel(latents, face_table, W1, b1, W2, b2):
    raise NotImplementedError("write your pallas kernel here")


# speedup vs baseline: 2.4012x; 2.4012x over previous
import functools
import math

import jax
import jax.numpy as jnp
from jax import lax
from jax.experimental import pallas as pl
from jax.experimental.pallas import tpu as pltpu

_F = 6
_E = 64


def _pe_kernel(ftT_ref, w1T_ref, b1_ref, w2T_ref, b2_ref, out_ref, scratch,
               *, H, W, B):
    HW = H * W
    f = pl.program_id(0)
    b = pl.program_id(1)

    @pl.when(jnp.logical_and(f == 0, b == 0))
    def _compute():
        j = lax.broadcasted_iota(jnp.int32, (1, HW), 1)
        x_row = (j % W).astype(jnp.float32) * (2.0 / (W - 1)) - 1.0
        y_row = (j // W).astype(jnp.float32) * (2.0 / (H - 1)) - 1.0
        w1T = w1T_ref[...]
        b1 = b1_ref[...]
        hT = w1T[:, 0:1] * x_row + w1T[:, 1:2] * y_row + b1
        hT = hT * 0.5 * (1.0 + lax.erf(hT * (1.0 / math.sqrt(2.0))))
        ceT = jax.lax.dot_general(
            w2T_ref[...], hT, (((1,), (0,)), ((), ())),
            preferred_element_type=jnp.float32,
            precision=lax.Precision.HIGHEST) + b2_ref[...]
        ftT = ftT_ref[...]
        for ff in range(_F):
            scratch[ff] = ceT + ftT[:, ff:ff + 1]

    out_ref[0] = scratch[f]


def kernel(latents, face_table, W1, b1, W2, b2):
    BF, _C, H, W = latents.shape
    B = BF // _F
    HW = H * W

    grid = (_F, B)
    out = pl.pallas_call(
        functools.partial(_pe_kernel, H=H, W=W, B=B),
        grid=grid,
        in_specs=[
            pl.BlockSpec((_E, _F), lambda f, b: (0, 0)),
            pl.BlockSpec((_E, 2), lambda f, b: (0, 0)),
            pl.BlockSpec((_E, 1), lambda f, b: (0, 0)),
            pl.BlockSpec((_E, _E), lambda f, b: (0, 0)),
            pl.BlockSpec((_E, 1), lambda f, b: (0, 0)),
        ],
        out_specs=pl.BlockSpec((1, _E, HW), lambda f, b: (b * _F + f, 0, 0)),
        out_shape=jax.ShapeDtypeStruct((BF, _E, HW), jnp.float32),
        scratch_shapes=[pltpu.VMEM((_F, _E, HW), jnp.float32)],
    )(face_table.T, W1.T, b1[:, None], W2.T, b2[:, None])
    return out.reshape(BF, _E, H, W)


# speedup vs baseline: 2.4084x; 1.0030x over previous
import functools
import math

import jax
import jax.numpy as jnp
from jax import lax
from jax.experimental import pallas as pl
from jax.experimental.pallas import tpu as pltpu

_F = 6
_E = 64


def _pe_kernel(ftT_ref, w1T_ref, b1_ref, w2T_ref, b2_ref, out_ref, scratch,
               sems, *, H, W, B):
    HW = H * W
    j = lax.broadcasted_iota(jnp.int32, (1, HW), 1)
    x_row = (j % W).astype(jnp.float32) * (2.0 / (W - 1)) - 1.0
    y_row = (j // W).astype(jnp.float32) * (2.0 / (H - 1)) - 1.0
    w1T = w1T_ref[...]
    hT = w1T[:, 0:1] * x_row + w1T[:, 1:2] * y_row + b1_ref[...]
    hT = hT * 0.5 * (1.0 + lax.erf(hT * (1.0 / math.sqrt(2.0))))
    ceT = jax.lax.dot_general(
        w2T_ref[...], hT, (((1,), (0,)), ((), ())),
        preferred_element_type=jnp.float32,
        precision=lax.Precision.HIGHEST) + b2_ref[...]
    ftT = ftT_ref[...]
    for f in range(_F):
        scratch[f] = ceT + ftT[:, f:f + 1]
        for b in range(B):
            pltpu.make_async_copy(
                scratch.at[f], out_ref.at[b * _F + f], sems.at[b * _F + f]
            ).start()
    for bf in range(B * _F):
        pltpu.make_async_copy(
            scratch.at[bf % _F], out_ref.at[bf], sems.at[bf]).wait()


def kernel(latents, face_table, W1, b1, W2, b2):
    BF, _C, H, W = latents.shape
    B = BF // _F
    HW = H * W

    out = pl.pallas_call(
        functools.partial(_pe_kernel, H=H, W=W, B=B),
        in_specs=[
            pl.BlockSpec(memory_space=pltpu.VMEM),
            pl.BlockSpec(memory_space=pltpu.VMEM),
            pl.BlockSpec(memory_space=pltpu.VMEM),
            pl.BlockSpec(memory_space=pltpu.VMEM),
            pl.BlockSpec(memory_space=pltpu.VMEM),
        ],
        out_specs=pl.BlockSpec(memory_space=pltpu.HBM),
        out_shape=jax.ShapeDtypeStruct((BF, _E, HW), jnp.float32),
        scratch_shapes=[
            pltpu.VMEM((_F, _E, HW), jnp.float32),
            pltpu.SemaphoreType.DMA((BF,)),
        ],
    )(face_table.T, W1.T, b1[:, None], W2.T, b2[:, None])
    return out.reshape(BF, _E, H, W)


# speedup vs baseline: 7.0849x; 2.9417x over previous
import functools
import math

import jax
import jax.numpy as jnp
from jax import lax
from jax.experimental import pallas as pl
from jax.experimental.pallas import tpu as pltpu

_F = 6
_E = 64


def _pe_kernel(ftT_ref, w1T_ref, b1_ref, w2T_ref, b2_ref, out_ref, scratch,
               sems, *, H, W, B):
    HW = H * W
    j = lax.broadcasted_iota(jnp.int32, (1, HW), 1)
    x_row = (j % W).astype(jnp.float32) * (2.0 / (W - 1)) - 1.0
    y_row = (j // W).astype(jnp.float32) * (2.0 / (H - 1)) - 1.0
    w1T = w1T_ref[...]
    hT = w1T[:, 0:1] * x_row + w1T[:, 1:2] * y_row + b1_ref[...]
    hT = hT * 0.5 * (1.0 + lax.erf(hT * (1.0 / math.sqrt(2.0))))
    ceT = jax.lax.dot_general(
        w2T_ref[...], hT, (((1,), (0,)), ((), ())),
        preferred_element_type=jnp.float32,
        precision=lax.Precision.HIGHEST) + b2_ref[...]
    ftT = ftT_ref[...]
    for f in range(_F):
        scratch[f] = (ceT + ftT[:, f:f + 1]).reshape(_E, H, W)
        for b in range(B):
            pltpu.make_async_copy(
                scratch.at[f], out_ref.at[b * _F + f], sems.at[b * _F + f]
            ).start()
    for bf in range(B * _F):
        pltpu.make_async_copy(
            scratch.at[bf % _F], out_ref.at[bf], sems.at[bf]).wait()


def kernel(latents, face_table, W1, b1, W2, b2):
    BF, _C, H, W = latents.shape
    B = BF // _F
    HW = H * W

    out = pl.pallas_call(
        functools.partial(_pe_kernel, H=H, W=W, B=B),
        in_specs=[
            pl.BlockSpec(memory_space=pltpu.VMEM),
            pl.BlockSpec(memory_space=pltpu.VMEM),
            pl.BlockSpec(memory_space=pltpu.VMEM),
            pl.BlockSpec(memory_space=pltpu.VMEM),
            pl.BlockSpec(memory_space=pltpu.VMEM),
        ],
        out_specs=pl.BlockSpec(memory_space=pltpu.HBM),
        out_shape=jax.ShapeDtypeStruct((BF, _E, H, W), jnp.float32),
        scratch_shapes=[
            pltpu.VMEM((_F, _E, H, W), jnp.float32),
            pltpu.SemaphoreType.DMA((BF,)),
        ],
    )(face_table.T, W1.T, b1[:, None], W2.T, b2[:, None])
    return out


# speedup vs baseline: 7.6785x; 1.0838x over previous
import functools
import math

import jax
import jax.numpy as jnp
from jax import lax
from jax.experimental import pallas as pl
from jax.experimental.pallas import tpu as pltpu

_F = 6
_E = 64
_NCH = 4


def _pe_kernel(ftT_ref, w1T_ref, b1_ref, w2T_ref, b2_ref, out_ref, scratch,
               sems, *, H, W, B):
    CH = H // _NCH
    CW = CH * W
    w1T = w1T_ref[...]
    ftT = ftT_ref[...]

    def copies(c, f):
        return [
            pltpu.make_async_copy(
                scratch.at[f, :, pl.ds(c * CH, CH), :],
                out_ref.at[b * _F + f, :, pl.ds(c * CH, CH), :],
                sems.at[b * _F + f])
            for b in range(B)
        ]

    for c in range(_NCH):
        j = lax.broadcasted_iota(jnp.int32, (1, CW), 1) + c * CW
        x_row = (j % W).astype(jnp.float32) * (2.0 / (W - 1)) - 1.0
        y_row = (j // W).astype(jnp.float32) * (2.0 / (H - 1)) - 1.0
        hT = w1T[:, 0:1] * x_row + w1T[:, 1:2] * y_row + b1_ref[...]
        hT = hT * 0.5 * (1.0 + lax.erf(hT * (1.0 / math.sqrt(2.0))))
        ceT = jax.lax.dot_general(
            w2T_ref[...], hT, (((1,), (0,)), ((), ())),
            preferred_element_type=jnp.float32,
            precision=lax.Precision.HIGHEST) + b2_ref[...]
        for f in range(_F):
            scratch[f, :, c * CH:(c + 1) * CH, :] = (
                ceT + ftT[:, f:f + 1]).reshape(_E, CH, W)
            for cp in copies(c, f):
                cp.start()
    for c in range(_NCH):
        for f in range(_F):
            for cp in copies(c, f):
                cp.wait()


def kernel(latents, face_table, W1, b1, W2, b2):
    BF, _C, H, W = latents.shape
    B = BF // _F

    out = pl.pallas_call(
        functools.partial(_pe_kernel, H=H, W=W, B=B),
        in_specs=[
            pl.BlockSpec(memory_space=pltpu.VMEM),
            pl.BlockSpec(memory_space=pltpu.VMEM),
            pl.BlockSpec(memory_space=pltpu.VMEM),
            pl.BlockSpec(memory_space=pltpu.VMEM),
            pl.BlockSpec(memory_space=pltpu.VMEM),
        ],
        out_specs=pl.BlockSpec(memory_space=pltpu.HBM),
        out_shape=jax.ShapeDtypeStruct((BF, _E, H, W), jnp.float32),
        scratch_shapes=[
            pltpu.VMEM((_F, _E, H, W), jnp.float32),
            pltpu.SemaphoreType.DMA((BF,)),
        ],
    )(face_table.T, W1.T, b1[:, None], W2.T, b2[:, None])
    return out


# speedup vs baseline: 7.8148x; 1.0177x over previous
import functools
import math

import jax
import jax.numpy as jnp
from jax import lax
from jax.experimental import pallas as pl
from jax.experimental.pallas import tpu as pltpu

_F = 6
_E = 64
_NCH = 8


def _pe_kernel(ftT_ref, w1T_ref, b1_ref, w2T_ref, b2_ref, out_ref, scratch,
               sems, *, H, W, B):
    CH = H // _NCH
    CW = CH * W
    w1T = w1T_ref[...]
    ftT = ftT_ref[...]

    def copies(c, f):
        return [
            pltpu.make_async_copy(
                scratch.at[f, :, pl.ds(c * CH, CH), :],
                out_ref.at[b * _F + f, :, pl.ds(c * CH, CH), :],
                sems.at[b * _F + f])
            for b in range(B)
        ]

    for c in range(_NCH):
        j = lax.broadcasted_iota(jnp.int32, (1, CW), 1) + c * CW
        x_row = (j % W).astype(jnp.float32) * (2.0 / (W - 1)) - 1.0
        y_row = (j // W).astype(jnp.float32) * (2.0 / (H - 1)) - 1.0
        hT = w1T[:, 0:1] * x_row + w1T[:, 1:2] * y_row + b1_ref[...]
        hT = hT * 0.5 * (1.0 + lax.erf(hT * (1.0 / math.sqrt(2.0))))
        ceT = jax.lax.dot_general(
            w2T_ref[...], hT, (((1,), (0,)), ((), ())),
            preferred_element_type=jnp.float32,
            precision=lax.Precision.HIGHEST) + b2_ref[...]
        for f in range(_F):
            scratch[f, :, c * CH:(c + 1) * CH, :] = (
                ceT + ftT[:, f:f + 1]).reshape(_E, CH, W)
            for cp in copies(c, f):
                cp.start()
    for c in range(_NCH):
        for f in range(_F):
            for cp in copies(c, f):
                cp.wait()


def kernel(latents, face_table, W1, b1, W2, b2):
    BF, _C, H, W = latents.shape
    B = BF // _F

    out = pl.pallas_call(
        functools.partial(_pe_kernel, H=H, W=W, B=B),
        in_specs=[
            pl.BlockSpec(memory_space=pltpu.VMEM),
            pl.BlockSpec(memory_space=pltpu.VMEM),
            pl.BlockSpec(memory_space=pltpu.VMEM),
            pl.BlockSpec(memory_space=pltpu.VMEM),
            pl.BlockSpec(memory_space=pltpu.VMEM),
        ],
        out_specs=pl.BlockSpec(memory_space=pltpu.HBM),
        out_shape=jax.ShapeDtypeStruct((BF, _E, H, W), jnp.float32),
        scratch_shapes=[
            pltpu.VMEM((_F, _E, H, W), jnp.float32),
            pltpu.SemaphoreType.DMA((BF,)),
        ],
    )(face_table.T, W1.T, b1[:, None], W2.T, b2[:, None])
    return out


# speedup vs baseline: 7.8702x; 1.0071x over previous
import functools
import math

import jax
import jax.numpy as jnp
from jax import lax
from jax.experimental import pallas as pl
from jax.experimental.pallas import tpu as pltpu

_F = 6
_E = 64
_NCH = 16


def _pe_kernel(ftT_ref, w1T_ref, b1_ref, w2T_ref, b2_ref, out_ref, scratch,
               sems, *, H, W, B):
    CH = H // _NCH
    CW = CH * W
    w1T = w1T_ref[...]
    ftT = ftT_ref[...]

    def copies(c, f):
        return [
            pltpu.make_async_copy(
                scratch.at[f, :, pl.ds(c * CH, CH), :],
                out_ref.at[b * _F + f, :, pl.ds(c * CH, CH), :],
                sems.at[b * _F + f])
            for b in range(B)
        ]

    for c in range(_NCH):
        j = lax.broadcasted_iota(jnp.int32, (1, CW), 1) + c * CW
        x_row = (j % W).astype(jnp.float32) * (2.0 / (W - 1)) - 1.0
        y_row = (j // W).astype(jnp.float32) * (2.0 / (H - 1)) - 1.0
        hT = w1T[:, 0:1] * x_row + w1T[:, 1:2] * y_row + b1_ref[...]
        hT = hT * 0.5 * (1.0 + lax.erf(hT * (1.0 / math.sqrt(2.0))))
        ceT = jax.lax.dot_general(
            w2T_ref[...], hT, (((1,), (0,)), ((), ())),
            preferred_element_type=jnp.float32,
            precision=lax.Precision.HIGHEST) + b2_ref[...]
        for f in range(_F):
            scratch[f, :, c * CH:(c + 1) * CH, :] = (
                ceT + ftT[:, f:f + 1]).reshape(_E, CH, W)
            for cp in copies(c, f):
                cp.start()
    for c in range(_NCH):
        for f in range(_F):
            for cp in copies(c, f):
                cp.wait()


def kernel(latents, face_table, W1, b1, W2, b2):
    BF, _C, H, W = latents.shape
    B = BF // _F

    out = pl.pallas_call(
        functools.partial(_pe_kernel, H=H, W=W, B=B),
        in_specs=[
            pl.BlockSpec(memory_space=pltpu.VMEM),
            pl.BlockSpec(memory_space=pltpu.VMEM),
            pl.BlockSpec(memory_space=pltpu.VMEM),
            pl.BlockSpec(memory_space=pltpu.VMEM),
            pl.BlockSpec(memory_space=pltpu.VMEM),
        ],
        out_specs=pl.BlockSpec(memory_space=pltpu.HBM),
        out_shape=jax.ShapeDtypeStruct((BF, _E, H, W), jnp.float32),
        scratch_shapes=[
            pltpu.VMEM((_F, _E, H, W), jnp.float32),
            pltpu.SemaphoreType.DMA((BF,)),
        ],
    )(face_table.T, W1.T, b1[:, None], W2.T, b2[:, None])
    return out
